# baseline (speedup 1.0000x reference)
import functools

import jax
import jax.numpy as jnp
from jax import lax
from jax.experimental import pallas as pl
from jax.experimental.pallas import tpu as pltpu, tpu_sc as plsc

N_NODES = 100000
N_EDGES = 6400000

NC = 2
NS = 16
NW = NC * NS
L = 16

CHUNK = 128
BLK = 8
NCH = N_EDGES // CHUNK
NBLK = NCH // BLK
BLK_BASE = NBLK // NW
BLK_EXTRA = NBLK % NW

N_PAD = 100352
SLICE_N = N_PAD // NS

_MESH = plsc.VectorSubcoreMesh(core_axis_name="c", subcore_axis_name="s")


def _rsqrt16(x):
    i = lax.bitcast_convert_type(x, jnp.int32)
    magic = jnp.full((L,), 0x5F3759DF, jnp.int32)
    one = jnp.full((L,), 1, jnp.int32)
    y = lax.bitcast_convert_type(magic - lax.shift_right_logical(i, one),
                                 jnp.float32)
    xh = x * jnp.float32(0.5)
    for _ in range(3):
        y = y * (jnp.float32(1.5) - xh * y * y)
    return y


def _worker_span(wid):
    b0 = wid * BLK_BASE + jnp.minimum(wid, BLK_EXTRA)
    nblk = jnp.where(wid < BLK_EXTRA, BLK_BASE + 1, BLK_BASE)
    return b0 * BLK, nblk


def _issue_loads(ew_hbm, idx_hbm, ew_v, src_v, dst_v, sem, r0):
    rows = pl.ds(r0, BLK)
    pltpu.async_copy(ew_hbm.at[rows], ew_v, sem)
    pltpu.async_copy(idx_hbm.at[0, rows], src_v, sem)
    pltpu.async_copy(idx_hbm.at[1, rows], dst_v, sem)


def _wait_loads(ew_hbm, idx_hbm, ew_v, src_v, dst_v, sem, r0):
    rows = pl.ds(r0, BLK)
    pltpu.make_async_copy(ew_hbm.at[rows], ew_v, sem).wait()
    pltpu.make_async_copy(idx_hbm.at[0, rows], src_v, sem).wait()
    pltpu.make_async_copy(idx_hbm.at[1, rows], dst_v, sem).wait()


@functools.partial(
    pl.kernel,
    mesh=_MESH,
    out_type=[
        jax.ShapeDtypeStruct((NC, N_PAD), jnp.float32),
        jax.ShapeDtypeStruct((NC, N_PAD), jnp.float32),
    ],
    scratch_types=[
        pltpu.VMEM_SHARED((N_PAD,), jnp.float32),
        pltpu.VMEM_SHARED((N_PAD,), jnp.float32),
        pltpu.VMEM((BLK, CHUNK), jnp.float32),
        pltpu.VMEM((BLK, CHUNK), jnp.int32),
        pltpu.VMEM((BLK, CHUNK), jnp.int32),
        pltpu.VMEM((BLK, CHUNK), jnp.float32),
        pltpu.VMEM((BLK, CHUNK), jnp.int32),
        pltpu.VMEM((BLK, CHUNK), jnp.int32),
        pltpu.VMEM((SLICE_N,), jnp.float32),
        pltpu.SemaphoreType.DMA,
        pltpu.SemaphoreType.DMA,
        pltpu.SemaphoreType.DMA,
    ],
)
def _deg_kernel(ew_hbm, idx_hbm, psrc_hbm, pdst_hbm,
                deg_src_sh, deg_dst_sh,
                ew_v0, src_v0, dst_v0, ew_v1, src_v1, dst_v1,
                zbuf, sem0, sem1, sem_sc):
    c = lax.axis_index("c")
    s = lax.axis_index("s")
    wid = s * NC + c
    set0 = (ew_v0, src_v0, dst_v0, sem0)
    set1 = (ew_v1, src_v1, dst_v1, sem1)

    def zset(i, carry):
        zbuf[pl.ds(i * L, L)] = jnp.zeros((L,), jnp.float32)
        return carry

    lax.fori_loop(0, SLICE_N // L, zset, 0)
    nsl = pl.ds(s * SLICE_N, SLICE_N)
    pltpu.sync_copy(zbuf, deg_src_sh.at[nsl])
    pltpu.sync_copy(zbuf, deg_dst_sh.at[nsl])
    plsc.subcore_barrier()

    base_row, nblk = _worker_span(wid)
    last_row = base_row + (nblk - 1) * BLK

    def addeps(ew_v):
        def step(j, inner):
            for i in range(CHUNK // L):
                sl = pl.ds(i * L, L)
                ew_v[j, sl] = ew_v[j, sl] + jnp.float32(1e-9)
            return inner

        lax.fori_loop(0, BLK, step, 0)

    def fire(ew_v, src_v, dst_v):
        copies = []
        for j in range(BLK):
            copies.append(pltpu.async_copy(
                ew_v.at[j], deg_src_sh.at[src_v.at[j]], sem_sc, add=True))
            copies.append(pltpu.async_copy(
                ew_v.at[j], deg_dst_sh.at[dst_v.at[j]], sem_sc, add=True))
        return copies

    def drain(copies):
        for cp in copies:
            cp.wait()

    _issue_loads(ew_hbm, idx_hbm, *set0, base_row)

    def body(g, carry):
        r0 = base_row + (2 * g) * BLK
        r1 = r0 + BLK
        r2 = jnp.minimum(r1 + BLK, last_row)
        _wait_loads(ew_hbm, idx_hbm, *set0, r0)
        _issue_loads(ew_hbm, idx_hbm, *set1, r1)
        addeps(ew_v0)
        cps0 = fire(ew_v0, src_v0, dst_v0)
        _wait_loads(ew_hbm, idx_hbm, *set1, r1)
        addeps(ew_v1)
        cps1 = fire(ew_v1, src_v1, dst_v1)
        drain(cps0)
        _issue_loads(ew_hbm, idx_hbm, *set0, r2)
        drain(cps1)
        return carry

    lax.fori_loop(0, nblk // 2, body, 0)
    _wait_loads(ew_hbm, idx_hbm, *set0, last_row)

    @pl.when(nblk % 2 == 1)
    def _():
        addeps(ew_v0)
        drain(fire(ew_v0, src_v0, dst_v0))

    plsc.subcore_barrier()
    pltpu.sync_copy(deg_src_sh.at[nsl], psrc_hbm.at[c, nsl])
    pltpu.sync_copy(deg_dst_sh.at[nsl], pdst_hbm.at[c, nsl])


@functools.partial(
    pl.kernel,
    mesh=_MESH,
    out_type=jax.ShapeDtypeStruct((NCH, CHUNK), jnp.float32),
    scratch_types=[
        pltpu.VMEM_SHARED((N_PAD,), jnp.float32),
        pltpu.VMEM_SHARED((N_PAD,), jnp.float32),
        pltpu.VMEM((SLICE_N,), jnp.float32),
        pltpu.VMEM((SLICE_N,), jnp.float32),
        pltpu.VMEM((BLK, CHUNK), jnp.float32),
        pltpu.VMEM((BLK, CHUNK), jnp.int32),
        pltpu.VMEM((BLK, CHUNK), jnp.int32),
        pltpu.VMEM((BLK, CHUNK), jnp.float32),
        pltpu.VMEM((BLK, CHUNK), jnp.int32),
        pltpu.VMEM((BLK, CHUNK), jnp.int32),
        pltpu.VMEM((BLK, CHUNK), jnp.float32),
        pltpu.VMEM((BLK, CHUNK), jnp.float32),
        pltpu.VMEM((BLK, CHUNK), jnp.float32),
        pltpu.VMEM((BLK, CHUNK), jnp.float32),
        pltpu.VMEM((BLK, CHUNK), jnp.float32),
        pltpu.VMEM((BLK, CHUNK), jnp.float32),
        pltpu.SemaphoreType.DMA,
        pltpu.SemaphoreType.DMA,
        pltpu.SemaphoreType.DMA,
        pltpu.SemaphoreType.DMA,
        pltpu.SemaphoreType.DMA,
    ],
)
def _apply_kernel(psrc_hbm, pdst_hbm, ew_hbm, idx_hbm, out_hbm,
                  nsrc_sh, ndst_sh, pa, pb,
                  ew_v0, src_v0, dst_v0, ew_v1, src_v1, dst_v1,
                  gs_v0, gd_v0, gs_v1, gd_v1, out_v0, out_v1,
                  sem0, sem1, sem_g, sem_o0, sem_o1):
    c = lax.axis_index("c")
    s = lax.axis_index("s")
    wid = s * NC + c
    nsl = pl.ds(s * SLICE_N, SLICE_N)
    set0 = (ew_v0, src_v0, dst_v0, sem0)
    set1 = (ew_v1, src_v1, dst_v1, sem1)

    for p_hbm, n_sh in ((psrc_hbm, nsrc_sh), (pdst_hbm, ndst_sh)):
        pltpu.sync_copy(p_hbm.at[0, nsl], pa)
        pltpu.sync_copy(p_hbm.at[1, nsl], pb)

        def cbody(i, carry):
            sl = pl.ds(i * L, L)
            pa[sl] = _rsqrt16(pa[sl] + pb[sl])
            return carry

        lax.fori_loop(0, SLICE_N // L, cbody, 0)
        pltpu.sync_copy(pa, n_sh.at[nsl])
    plsc.subcore_barrier()

    base_row, nblk = _worker_span(wid)
    last_row = base_row + (nblk - 1) * BLK

    def fire(src_v, dst_v, gs_v, gd_v):
        copies = []
        for j in range(BLK):
            copies.append(pltpu.async_copy(
                nsrc_sh.at[src_v.at[j]], gs_v.at[j], sem_g))
            copies.append(pltpu.async_copy(
                ndst_sh.at[dst_v.at[j]], gd_v.at[j], sem_g))
        return copies

    def finish(first, copies, ew_v, gs_v, gd_v, out_v, sem_o, r):
        for cp in copies:
            cp.wait()

        @pl.when(jnp.logical_not(first))
        def _():
            pltpu.make_async_copy(out_v, out_hbm.at[pl.ds(r, BLK)], sem_o).wait()

        def fma(j, inner):
            for i in range(CHUNK // L):
                sl = pl.ds(i * L, L)
                out_v[j, sl] = (gs_v[j, sl] * gd_v[j, sl]
                                * (ew_v[j, sl] + jnp.float32(1e-9)))
            return inner

        lax.fori_loop(0, BLK, fma, 0)
        pltpu.async_copy(out_v, out_hbm.at[pl.ds(r, BLK)], sem_o)

    _issue_loads(ew_hbm, idx_hbm, *set0, base_row)

    def body(g, carry):
        r0 = base_row + (2 * g) * BLK
        r1 = r0 + BLK
        r2 = jnp.minimum(r1 + BLK, last_row)
        _wait_loads(ew_hbm, idx_hbm, *set0, r0)
        _issue_loads(ew_hbm, idx_hbm, *set1, r1)
        cps0 = fire(src_v0, dst_v0, gs_v0, gd_v0)
        _wait_loads(ew_hbm, idx_hbm, *set1, r1)
        cps1 = fire(src_v1, dst_v1, gs_v1, gd_v1)
        finish(g == 0, cps0, ew_v0, gs_v0, gd_v0, out_v0, sem_o0, r0)
        _issue_loads(ew_hbm, idx_hbm, *set0, r2)
        finish(g == 0, cps1, ew_v1, gs_v1, gd_v1, out_v1, sem_o1, r1)
        return carry

    lax.fori_loop(0, nblk // 2, body, 0)
    _wait_loads(ew_hbm, idx_hbm, *set0, last_row)

    @pl.when(nblk % 2 == 1)
    def _():
        finish(False, fire(src_v0, dst_v0, gs_v0, gd_v0),
               ew_v0, gs_v0, gd_v0, out_v0, sem_o0, last_row)

    pltpu.make_async_copy(out_v0, out_hbm.at[pl.ds(last_row, BLK)], sem_o0).wait()
    pltpu.make_async_copy(out_v1, out_hbm.at[pl.ds(last_row, BLK)], sem_o1).wait()


def kernel(edge_weight, edge_index):
    ew2 = edge_weight.astype(jnp.float32).reshape(NCH, CHUNK)
    idx3 = edge_index.astype(jnp.int32).reshape(2, NCH, CHUNK)
    psrc, pdst = _deg_kernel(ew2, idx3)
    out2 = _apply_kernel(psrc, pdst, ew2, idx3)
    return out2.reshape(-1)


# speedup vs baseline: 1.1575x; 1.1575x over previous
import functools

import jax
import jax.numpy as jnp
from jax import lax
from jax.experimental import pallas as pl
from jax.experimental.pallas import tpu as pltpu, tpu_sc as plsc

N_NODES = 100000
N_EDGES = 6400000

NC = 2
NS = 16
NW = NC * NS
L = 16

CHUNK = 128
BLK = 8
BE = BLK * CHUNK
NCH = N_EDGES // CHUNK
NBLK = NCH // BLK
BLK_BASE = NBLK // NW
BLK_EXTRA = NBLK % NW

N_PAD = 100352
SLICE_N = N_PAD // NS

_MESH = plsc.VectorSubcoreMesh(core_axis_name="c", subcore_axis_name="s")


def _rsqrt16(x):
    i = lax.bitcast_convert_type(x, jnp.int32)
    magic = jnp.full((L,), 0x5F3759DF, jnp.int32)
    one = jnp.full((L,), 1, jnp.int32)
    y = lax.bitcast_convert_type(magic - lax.shift_right_logical(i, one),
                                 jnp.float32)
    xh = x * jnp.float32(0.5)
    for _ in range(3):
        y = y * (jnp.float32(1.5) - xh * y * y)
    return y


def _worker_span(wid):
    b0 = wid * BLK_BASE + jnp.minimum(wid, BLK_EXTRA)
    nblk = jnp.where(wid < BLK_EXTRA, BLK_BASE + 1, BLK_BASE)
    return b0 * BLK, nblk


def _issue_loads(ew_hbm, idx_hbm, ew_v, src_v, dst_v, sem, r0):
    q = pl.ds(r0 * CHUNK, BE)
    pltpu.async_copy(ew_hbm.at[q], ew_v, sem)
    pltpu.async_copy(idx_hbm.at[0, q], src_v, sem)
    pltpu.async_copy(idx_hbm.at[1, q], dst_v, sem)


def _wait_loads(ew_hbm, idx_hbm, ew_v, src_v, dst_v, sem, r0):
    q = pl.ds(r0 * CHUNK, BE)
    pltpu.make_async_copy(ew_hbm.at[q], ew_v, sem).wait()
    pltpu.make_async_copy(idx_hbm.at[0, q], src_v, sem).wait()
    pltpu.make_async_copy(idx_hbm.at[1, q], dst_v, sem).wait()


@functools.partial(
    pl.kernel,
    mesh=_MESH,
    out_type=[
        jax.ShapeDtypeStruct((NC, N_PAD), jnp.float32),
        jax.ShapeDtypeStruct((NC, N_PAD), jnp.float32),
    ],
    scratch_types=[
        pltpu.VMEM_SHARED((N_PAD,), jnp.float32),
        pltpu.VMEM_SHARED((N_PAD,), jnp.float32),
        pltpu.VMEM((BE,), jnp.float32),
        pltpu.VMEM((BE,), jnp.int32),
        pltpu.VMEM((BE,), jnp.int32),
        pltpu.VMEM((BE,), jnp.float32),
        pltpu.VMEM((BE,), jnp.int32),
        pltpu.VMEM((BE,), jnp.int32),
        pltpu.VMEM((SLICE_N,), jnp.float32),
        pltpu.SemaphoreType.DMA,
        pltpu.SemaphoreType.DMA,
        pltpu.SemaphoreType.DMA,
    ],
)
def _deg_kernel(ew_hbm, idx_hbm, psrc_hbm, pdst_hbm,
                deg_src_sh, deg_dst_sh,
                ew_v0, src_v0, dst_v0, ew_v1, src_v1, dst_v1,
                zbuf, sem0, sem1, sem_sc):
    c = lax.axis_index("c")
    s = lax.axis_index("s")
    wid = s * NC + c
    set0 = (ew_v0, src_v0, dst_v0, sem0)
    set1 = (ew_v1, src_v1, dst_v1, sem1)

    def zset(i, carry):
        zbuf[pl.ds(i * L, L)] = jnp.zeros((L,), jnp.float32)
        return carry

    lax.fori_loop(0, SLICE_N // L, zset, 0)
    nsl = pl.ds(s * SLICE_N, SLICE_N)
    pltpu.sync_copy(zbuf, deg_src_sh.at[nsl])
    pltpu.sync_copy(zbuf, deg_dst_sh.at[nsl])
    plsc.subcore_barrier()

    base_row, nblk = _worker_span(wid)
    last_row = base_row + (nblk - 1) * BLK

    def process(ew_v, src_v, dst_v):
        def addeps(j, inner):
            for i in range(CHUNK // L):
                sl = pl.ds(j * CHUNK + i * L, L)
                ew_v[sl] = ew_v[sl] + jnp.float32(1e-9)
            return inner

        lax.fori_loop(0, BLK, addeps, 0)
        c1 = pltpu.async_copy(ew_v, deg_src_sh.at[src_v], sem_sc, add=True)
        c2 = pltpu.async_copy(ew_v, deg_dst_sh.at[dst_v], sem_sc, add=True)
        c1.wait()
        c2.wait()

    _issue_loads(ew_hbm, idx_hbm, *set0, base_row)

    def body(g, carry):
        r0 = base_row + (2 * g) * BLK
        r1 = r0 + BLK
        r2 = jnp.minimum(r1 + BLK, last_row)
        _wait_loads(ew_hbm, idx_hbm, *set0, r0)
        _issue_loads(ew_hbm, idx_hbm, *set1, r1)
        process(ew_v0, src_v0, dst_v0)
        _wait_loads(ew_hbm, idx_hbm, *set1, r1)
        _issue_loads(ew_hbm, idx_hbm, *set0, r2)
        process(ew_v1, src_v1, dst_v1)
        return carry

    lax.fori_loop(0, nblk // 2, body, 0)
    _wait_loads(ew_hbm, idx_hbm, *set0, last_row)

    @pl.when(nblk % 2 == 1)
    def _():
        process(ew_v0, src_v0, dst_v0)

    plsc.subcore_barrier()
    pltpu.sync_copy(deg_src_sh.at[nsl], psrc_hbm.at[c, nsl])
    pltpu.sync_copy(deg_dst_sh.at[nsl], pdst_hbm.at[c, nsl])


@functools.partial(
    pl.kernel,
    mesh=_MESH,
    out_type=jax.ShapeDtypeStruct((N_EDGES,), jnp.float32),
    scratch_types=[
        pltpu.VMEM_SHARED((N_PAD,), jnp.float32),
        pltpu.VMEM_SHARED((N_PAD,), jnp.float32),
        pltpu.VMEM((SLICE_N,), jnp.float32),
        pltpu.VMEM((SLICE_N,), jnp.float32),
        pltpu.VMEM((BE,), jnp.float32),
        pltpu.VMEM((BE,), jnp.int32),
        pltpu.VMEM((BE,), jnp.int32),
        pltpu.VMEM((BE,), jnp.float32),
        pltpu.VMEM((BE,), jnp.int32),
        pltpu.VMEM((BE,), jnp.int32),
        pltpu.VMEM((BE,), jnp.float32),
        pltpu.VMEM((BE,), jnp.float32),
        pltpu.VMEM((BE,), jnp.float32),
        pltpu.VMEM((BE,), jnp.float32),
        pltpu.VMEM((BE,), jnp.float32),
        pltpu.VMEM((BE,), jnp.float32),
        pltpu.SemaphoreType.DMA,
        pltpu.SemaphoreType.DMA,
        pltpu.SemaphoreType.DMA,
        pltpu.SemaphoreType.DMA,
        pltpu.SemaphoreType.DMA,
    ],
)
def _apply_kernel(psrc_hbm, pdst_hbm, ew_hbm, idx_hbm, out_hbm,
                  nsrc_sh, ndst_sh, pa, pb,
                  ew_v0, src_v0, dst_v0, ew_v1, src_v1, dst_v1,
                  gs_v0, gd_v0, gs_v1, gd_v1, out_v0, out_v1,
                  sem0, sem1, sem_g, sem_o0, sem_o1):
    c = lax.axis_index("c")
    s = lax.axis_index("s")
    wid = s * NC + c
    nsl = pl.ds(s * SLICE_N, SLICE_N)
    set0 = (ew_v0, src_v0, dst_v0, sem0)
    set1 = (ew_v1, src_v1, dst_v1, sem1)

    for p_hbm, n_sh in ((psrc_hbm, nsrc_sh), (pdst_hbm, ndst_sh)):
        pltpu.sync_copy(p_hbm.at[0, nsl], pa)
        pltpu.sync_copy(p_hbm.at[1, nsl], pb)

        def cbody(i, carry):
            sl = pl.ds(i * L, L)
            pa[sl] = _rsqrt16(pa[sl] + pb[sl])
            return carry

        lax.fori_loop(0, SLICE_N // L, cbody, 0)
        pltpu.sync_copy(pa, n_sh.at[nsl])
    plsc.subcore_barrier()

    base_row, nblk = _worker_span(wid)
    last_row = base_row + (nblk - 1) * BLK

    def process(first, ew_v, src_v, dst_v, gs_v, gd_v, out_v, sem_o, r):
        c1 = pltpu.async_copy(nsrc_sh.at[src_v], gs_v, sem_g)
        c2 = pltpu.async_copy(ndst_sh.at[dst_v], gd_v, sem_g)
        c1.wait()
        c2.wait()

        q = pl.ds(r * CHUNK, BE)

        @pl.when(jnp.logical_not(first))
        def _():
            pltpu.make_async_copy(out_v, out_hbm.at[q], sem_o).wait()

        def fma(j, inner):
            for i in range(CHUNK // L):
                sl = pl.ds(j * CHUNK + i * L, L)
                out_v[sl] = (gs_v[sl] * gd_v[sl]
                             * (ew_v[sl] + jnp.float32(1e-9)))
            return inner

        lax.fori_loop(0, BLK, fma, 0)
        pltpu.async_copy(out_v, out_hbm.at[q], sem_o)

    _issue_loads(ew_hbm, idx_hbm, *set0, base_row)

    def body(g, carry):
        r0 = base_row + (2 * g) * BLK
        r1 = r0 + BLK
        r2 = jnp.minimum(r1 + BLK, last_row)
        _wait_loads(ew_hbm, idx_hbm, *set0, r0)
        _issue_loads(ew_hbm, idx_hbm, *set1, r1)
        process(g == 0, ew_v0, src_v0, dst_v0, gs_v0, gd_v0, out_v0, sem_o0, r0)
        _wait_loads(ew_hbm, idx_hbm, *set1, r1)
        _issue_loads(ew_hbm, idx_hbm, *set0, r2)
        process(g == 0, ew_v1, src_v1, dst_v1, gs_v1, gd_v1, out_v1, sem_o1, r1)
        return carry

    lax.fori_loop(0, nblk // 2, body, 0)
    _wait_loads(ew_hbm, idx_hbm, *set0, last_row)

    @pl.when(nblk % 2 == 1)
    def _():
        process(False, ew_v0, src_v0, dst_v0, gs_v0, gd_v0, out_v0, sem_o0,
                last_row)

    qlast = pl.ds(last_row * CHUNK, BE)
    pltpu.make_async_copy(out_v0, out_hbm.at[qlast], sem_o0).wait()
    pltpu.make_async_copy(out_v1, out_hbm.at[qlast], sem_o1).wait()


def kernel(edge_weight, edge_index):
    ew = edge_weight.astype(jnp.float32)
    idx = edge_index.astype(jnp.int32)
    psrc, pdst = _deg_kernel(ew, idx)
    return _apply_kernel(psrc, pdst, ew, idx)
